# baseline (speedup 1.0000x reference)
import functools

import jax
import jax.numpy as jnp
from jax import lax
from jax.experimental import pallas as pl
from jax.experimental.pallas import tpu as pltpu
from jax.experimental.pallas import tpu_sc as plsc

N, D, G = 100000, 128, 512
NC, NS, L = 2, 16, 16
NW = NC * NS
CH = 160
NCHUNK = N // CH
VPR = D // L


def _sc_body(feats_hbm, seg_hbm, smask_hbm, w_hbm, b_hbm, out_hbm,
             feats_v0, seg_v0, smask_v0, feats_v1, seg_v1, smask_v1,
             accum_v, w_v, b_v, sem0, sem1):
    cid = lax.axis_index("c")
    sid = lax.axis_index("s")
    wid = sid * NC + cid

    zero16 = jnp.zeros((L,), jnp.float32)

    def zrow(r, carry):
        for j in range(VPR):
            accum_v[r, pl.ds(L * j, L)] = zero16
        return carry

    lax.fori_loop(0, G, zrow, 0)

    pltpu.sync_copy(w_hbm, w_v)
    pltpu.sync_copy(b_hbm, b_v.at[pl.ds(0, 1)])
    wv = [w_v[pl.ds(L * j, L)] for j in range(VPR)]
    b_s = b_v[pl.ds(0, L)][0]

    nk = (NCHUNK - wid + NW - 1) // NW

    bufs = ((feats_v0, seg_v0, smask_v0, sem0),
            (feats_v1, seg_v1, smask_v1, sem1))

    def issue(k, p):
        base = (wid + NW * k) * CH
        fb, sb, mb, sem = bufs[p]
        pltpu.async_copy(feats_hbm.at[pl.ds(base, CH), :], fb, sem)
        pltpu.async_copy(seg_hbm.at[pl.ds(base, CH)], sb, sem)
        pltpu.async_copy(smask_hbm.at[pl.ds(base, CH)], mb, sem)

    def drain(p):
        fb, sb, mb, sem = bufs[p]
        pltpu.make_async_copy(feats_hbm.at[pl.ds(0, CH), :], fb, sem).wait()
        pltpu.make_async_copy(seg_hbm.at[pl.ds(0, CH)], sb, sem).wait()
        pltpu.make_async_copy(smask_hbm.at[pl.ds(0, CH)], mb, sem).wait()

    def process(p):
        fb, sb, mb, _ = bufs[p]

        def group_body(g, c2):
            seg16 = sb[pl.ds(L * g, L)]
            sm16 = mb[pl.ds(L * g, L)]
            PD = 4
            held, dots = {}, {}
            for t in range(L + PD):
                if t < L:
                    r = L * g + t
                    v = [fb[r, pl.ds(L * j, L)] for j in range(VPR)]
                    held[t] = v
                    m = [v[j] * wv[j] for j in range(VPR)]
                    while len(m) > 1:
                        m = [m[2 * q] + m[2 * q + 1]
                             for q in range(len(m) // 2)]
                    dots[t] = jnp.sum(m[0]) + b_s
                if t >= PD:
                    i = t - PD
                    sv = jnp.full((L,), dots.pop(i), jnp.float32)
                    sig = 1.0 / (1.0 + jnp.exp(-sv))
                    gv = jnp.full((L,), sig[0] * sm16[i], jnp.float32)
                    seg = seg16[i]
                    v = held.pop(i)
                    ts = [gv * v[j] for j in range(VPR)]
                    for j in range(VPR):
                        plsc.addupdate(accum_v.at[seg, pl.ds(L * j, L)],
                                       ts[j])
            return c2

        lax.fori_loop(0, CH // L, group_body, 0)

    issue(0, 0)

    def pair_body(kk, carry):
        k0 = 2 * kk
        k1 = k0 + 1

        @pl.when(k1 < nk)
        def _():
            issue(k1, 1)

        drain(0)
        process(0)

        @pl.when(k1 + 1 < nk)
        def _():
            issue(k1 + 1, 0)

        @pl.when(k1 < nk)
        def _():
            drain(1)
            process(1)

        return carry

    lax.fori_loop(0, (NCHUNK + 2 * NW - 1) // (2 * NW), pair_body, 0)

    pltpu.sync_copy(accum_v, out_hbm.at[wid])


_sc_call = pl.kernel(
    _sc_body,
    out_type=jax.ShapeDtypeStruct((NW, G, D), jnp.float32),
    mesh=plsc.VectorSubcoreMesh(core_axis_name="c", subcore_axis_name="s"),
    compiler_params=pltpu.CompilerParams(needs_layout_passes=False),
    scratch_types=[
        pltpu.VMEM((CH, D), jnp.float32),
        pltpu.VMEM((CH,), jnp.int32),
        pltpu.VMEM((CH,), jnp.float32),
        pltpu.VMEM((CH, D), jnp.float32),
        pltpu.VMEM((CH,), jnp.int32),
        pltpu.VMEM((CH,), jnp.float32),
        pltpu.VMEM((G, D), jnp.float32),
        pltpu.VMEM((D,), jnp.float32),
        pltpu.VMEM((L,), jnp.float32),
        pltpu.SemaphoreType.DMA,
        pltpu.SemaphoreType.DMA,
    ],
)


def _add_body(p_ref, o_ref):
    o_ref[...] = jnp.sum(p_ref[...], axis=0)


_tc_add = pl.pallas_call(
    _add_body,
    out_shape=jax.ShapeDtypeStruct((G, D), jnp.float32),
)


def kernel(feats, smask, segment_ids, W, b):
    seg32 = segment_ids.astype(jnp.int32)
    smask1 = smask.reshape((N,))
    w1 = W.reshape((D,))
    partials = _sc_call(feats, seg32, smask1, w1, b)
    return _tc_add(partials)


# speedup vs baseline: 1.3317x; 1.3317x over previous
import functools

import jax
import jax.numpy as jnp
from jax import lax
from jax.experimental import pallas as pl
from jax.experimental.pallas import tpu as pltpu
from jax.experimental.pallas import tpu_sc as plsc

N, D, G = 100000, 128, 512
NC, NS, L = 2, 16, 16
NW = NC * NS
CH = 160
NCHUNK = N // CH
VPR = D // L


def _sc_body(feats_hbm, seg_hbm, smask_hbm, w_hbm, b_hbm, out_hbm,
             feats_v0, seg_v0, smask_v0, feats_v1, seg_v1, smask_v1,
             accum_v, w_v, b_v, sem0, sem1):
    cid = lax.axis_index("c")
    sid = lax.axis_index("s")
    wid = sid * NC + cid

    nk = (NCHUNK - wid + NW - 1) // NW

    bufs = ((feats_v0, seg_v0, smask_v0, sem0),
            (feats_v1, seg_v1, smask_v1, sem1))

    def issue(k, p):
        base = (wid + NW * k) * CH
        fb, sb, mb, sem = bufs[p]
        pltpu.async_copy(feats_hbm.at[pl.ds(base, CH), :], fb, sem)
        pltpu.async_copy(seg_hbm.at[pl.ds(base, CH)], sb, sem)
        pltpu.async_copy(smask_hbm.at[pl.ds(base, CH)], mb, sem)

    issue(0, 0)

    zero16 = jnp.zeros((L,), jnp.float32)

    def zrow(r, carry):
        for j in range(VPR):
            accum_v[r, pl.ds(L * j, L)] = zero16
        return carry

    lax.fori_loop(0, G, zrow, 0)

    pltpu.sync_copy(w_hbm, w_v)
    pltpu.sync_copy(b_hbm, b_v.at[pl.ds(0, 1)])
    wv = [w_v[pl.ds(L * j, L)] for j in range(VPR)]
    b_s = b_v[pl.ds(0, L)][0]

    def drain(p):
        fb, sb, mb, sem = bufs[p]
        pltpu.make_async_copy(feats_hbm.at[pl.ds(0, CH), :], fb, sem).wait()
        pltpu.make_async_copy(seg_hbm.at[pl.ds(0, CH)], sb, sem).wait()
        pltpu.make_async_copy(smask_hbm.at[pl.ds(0, CH)], mb, sem).wait()

    def process(p):
        fb, sb, mb, _ = bufs[p]

        def group_body(g, c2):
            seg16 = sb[pl.ds(L * g, L)]
            sm16 = mb[pl.ds(L * g, L)]
            ss = []
            for i in range(L):
                r = L * g + i
                v = [fb[r, pl.ds(L * j, L)] for j in range(VPR)]
                m = [v[j] * wv[j] for j in range(VPR)]
                while len(m) > 1:
                    m = [m[2 * t] + m[2 * t + 1] for t in range(len(m) // 2)]
                ss.append(jnp.sum(m[0]) + b_s)
            gates = []
            for i in range(L):
                sv = jnp.full((L,), ss[i], jnp.float32)
                sig = 1.0 / (1.0 + jnp.exp(-sv))
                gates.append(sig[0] * sm16[i])
            for i in range(L):
                r = L * g + i
                seg = seg16[i]
                gv = jnp.full((L,), gates[i], jnp.float32)
                ts = [gv * fb[r, pl.ds(L * j, L)] for j in range(VPR)]
                for j in range(VPR):
                    plsc.addupdate(accum_v.at[seg, pl.ds(L * j, L)], ts[j])
            return c2

        lax.fori_loop(0, CH // L, group_body, 0)

    def pair_body(kk, carry):
        k0 = 2 * kk
        k1 = k0 + 1

        @pl.when(k1 < nk)
        def _():
            issue(k1, 1)

        drain(0)
        process(0)

        @pl.when(k1 + 1 < nk)
        def _():
            issue(k1 + 1, 0)

        @pl.when(k1 < nk)
        def _():
            drain(1)
            process(1)

        return carry

    lax.fori_loop(0, (NCHUNK + 2 * NW - 1) // (2 * NW), pair_body, 0)

    pltpu.sync_copy(accum_v, out_hbm.at[wid])


_sc_call = pl.kernel(
    _sc_body,
    out_type=jax.ShapeDtypeStruct((NW, G, D), jnp.float32),
    mesh=plsc.VectorSubcoreMesh(core_axis_name="c", subcore_axis_name="s"),
    compiler_params=pltpu.CompilerParams(needs_layout_passes=False),
    scratch_types=[
        pltpu.VMEM((CH, D), jnp.float32),
        pltpu.VMEM((CH,), jnp.int32),
        pltpu.VMEM((CH,), jnp.float32),
        pltpu.VMEM((CH, D), jnp.float32),
        pltpu.VMEM((CH,), jnp.int32),
        pltpu.VMEM((CH,), jnp.float32),
        pltpu.VMEM((G, D), jnp.float32),
        pltpu.VMEM((D,), jnp.float32),
        pltpu.VMEM((L,), jnp.float32),
        pltpu.SemaphoreType.DMA,
        pltpu.SemaphoreType.DMA,
    ],
)


def _add_body(p_ref, o_ref):
    o_ref[...] = jnp.sum(p_ref[...], axis=0)


_tc_add = pl.pallas_call(
    _add_body,
    out_shape=jax.ShapeDtypeStruct((G, D), jnp.float32),
)


def kernel(feats, smask, segment_ids, W, b):
    seg32 = segment_ids.astype(jnp.int32)
    smask1 = smask.reshape((N,))
    w1 = W.reshape((D,))
    partials = _sc_call(feats, seg32, smask1, w1, b)
    return _tc_add(partials)
